# baseline (speedup 1.0000x reference)
import functools

import jax
import jax.numpy as jnp
from jax import lax
from jax.experimental import pallas as pl
from jax.experimental.pallas import tpu as pltpu
from jax.experimental.pallas import tpu_sc as plsc

N = 10000
D = 128
HID = 8
HEADS = 8
OUT = 7
E = 320000

NC, NS, L = 2, 16, 16
NW = NC * NS
CHUNK = 128
NP = 10112
RPT = NP // NS
ETOT = E + N
CPT = 82
EP = NW * CHUNK * CPT
S2 = 2
C2 = S2 * CHUNK
CPT2 = CPT // S2

F1 = HEADS * HID
W1R = F1 + L


def _dense1_body(x_ref, m1_ref, wad_ref, hbf_ref, asil_ref, dstt_ref,
                 bs_ref, bd_ref):
    p = jnp.dot(x_ref[...], m1_ref[...], preferred_element_type=jnp.float32)
    q = jnp.dot(x_ref[...], wad_ref[...], preferred_element_type=jnp.float32)
    hbf_ref[...] = p[:, 0:F1].astype(jnp.bfloat16)
    asil_ref[...] = p[:, F1:W1R]
    dstt_ref[...] = q
    bs_ref[0, :, :] = jnp.max(p[:, F1:W1R], axis=0, keepdims=True)
    bd_ref[0, :, :] = jnp.max(q, axis=0, keepdims=True)


def _dense1(x_p, M1, WAd):
    rb = 1264
    grid = NP // rb
    return pl.pallas_call(
        _dense1_body,
        grid=(grid,),
        in_specs=[
            pl.BlockSpec((rb, D), lambda i: (i, 0)),
            pl.BlockSpec((D, W1R), lambda i: (0, 0)),
            pl.BlockSpec((D, L), lambda i: (0, 0)),
        ],
        out_specs=[
            pl.BlockSpec((rb, F1), lambda i: (i, 0)),
            pl.BlockSpec((rb, L), lambda i: (i, 0)),
            pl.BlockSpec((rb, L), lambda i: (i, 0)),
            pl.BlockSpec((1, 1, L), lambda i: (i, 0, 0)),
            pl.BlockSpec((1, 1, L), lambda i: (i, 0, 0)),
        ],
        out_shape=[
            jax.ShapeDtypeStruct((NP, F1), jnp.bfloat16),
            jax.ShapeDtypeStruct((NP, L), jnp.float32),
            jax.ShapeDtypeStruct((NP, L), jnp.float32),
            jax.ShapeDtypeStruct((grid, 1, L), jnp.float32),
            jax.ShapeDtypeStruct((grid, 1, L), jnp.float32),
        ],
    )(x_p, M1, WAd)


def _edge1_body(si3, di3, srct_hbm, dstt_hbm, cv_hbm, z_hbm, out_hbm,
                sidx_all, didx_all, srows0, srows1, drows0, drows1,
                orows0, orows1, cv, acc, gs0, gs1, gd0, gd1, sc0, sc1):
    ci = lax.axis_index("c")
    sid = lax.axis_index("s")
    wid = sid * NC + ci

    pltpu.sync_copy(z_hbm, acc.at[pl.ds(sid * RPT, RPT)])
    pltpu.sync_copy(cv_hbm, cv)
    pltpu.sync_copy(si3.at[wid], sidx_all)
    pltpu.sync_copy(di3.at[wid], didx_all)
    plsc.subcore_barrier()

    srows = (srows0, srows1)
    drows = (drows0, drows1)
    orows = (orows0, orows1)
    gs = (gs0, gs1)
    gd = (gd0, gd1)
    sc = (sc0, sc1)
    cvr = cv[pl.ds(0, L)]

    def start_g(c, b):
        pltpu.async_copy(srct_hbm.at[sidx_all.at[c]], srows[b], gs[b])
        pltpu.async_copy(dstt_hbm.at[didx_all.at[c]], drows[b], gd[b])

    def wait_g(c, b):
        pltpu.make_async_copy(srct_hbm.at[sidx_all.at[c]], srows[b],
                              gs[b]).wait()
        pltpu.make_async_copy(dstt_hbm.at[didx_all.at[c]], drows[b],
                              gd[b]).wait()

    def start_s(c, b):
        pltpu.async_copy(orows[b], acc.at[didx_all.at[c]], sc[b], add=True)

    def wait_s(c, b):
        pltpu.make_async_copy(orows[b], acc.at[didx_all.at[c]], sc[b]).wait()

    def compute(b):
        sr, dr, orr = srows[b], drows[b], orows[b]

        def edge_body(i, c2):
            asil = plsc.bitcast(sr[i, pl.ds(0, 2 * L)], jnp.float32)
            t = asil + dr[i, pl.ds(0, L)]
            t = jnp.where(t > 0, t, t * jnp.float32(0.2))
            ex = jnp.exp(t - cvr)
            for v in range(2):
                hv = sr[i, pl.ds(2 * L + 2 * v * L, 2 * L)]
                ha, hb = plsc.unpack(hv, format=plsc.PackFormat.INTERLEAVED)
                orr[i, pl.ds(2 * v * L, L)] = ha * ex
                orr[i, pl.ds(2 * v * L + L, L)] = hb * ex
            orr[i, pl.ds(F1, L)] = ex
            return c2

        lax.fori_loop(0, CHUNK, edge_body, 0, unroll=4)

    start_g(0, 0)
    start_g(1, 1)
    wait_g(0, 0)
    compute(0)
    start_s(0, 0)
    start_g(2, 0)
    wait_g(1, 1)
    compute(1)
    start_s(1, 1)
    start_g(3, 1)

    def pair(k, carry):
        ca = 2 * k
        cb = 2 * k + 1
        wait_g(ca, 0)
        wait_s(ca - 2, 0)
        compute(0)
        start_s(ca, 0)

        @pl.when(ca + 2 < CPT)
        def _():
            start_g(ca + 2, 0)

        wait_g(cb, 1)
        wait_s(cb - 2, 1)
        compute(1)
        start_s(cb, 1)

        @pl.when(cb + 2 < CPT)
        def _():
            start_g(cb + 2, 1)

        return carry

    lax.fori_loop(1, CPT // 2, pair, 0)
    wait_s(CPT - 2, 0)
    wait_s(CPT - 1, 1)
    plsc.subcore_barrier()
    pltpu.sync_copy(acc.at[pl.ds(sid * RPT, RPT)],
                    out_hbm.at[ci, pl.ds(sid * RPT, RPT)])


@functools.cache
def _edge1():
  return pl.kernel(
    _edge1_body,
    out_type=jax.ShapeDtypeStruct((NC, NP, W1R), jnp.float32),
    compiler_params=pltpu.CompilerParams(use_tc_tiling_on_sc=False,
                                         needs_layout_passes=False),
    mesh=plsc.VectorSubcoreMesh(core_axis_name="c", subcore_axis_name="s",
                                num_cores=NC, num_subcores=NS),
    scratch_types=[
        pltpu.VMEM((CPT, CHUNK), jnp.int32),
        pltpu.VMEM((CPT, CHUNK), jnp.int32),
        pltpu.VMEM((CHUNK, 6 * L), jnp.bfloat16),
        pltpu.VMEM((CHUNK, 6 * L), jnp.bfloat16),
        pltpu.VMEM((CHUNK, L), jnp.float32),
        pltpu.VMEM((CHUNK, L), jnp.float32),
        pltpu.VMEM((CHUNK, W1R), jnp.float32),
        pltpu.VMEM((CHUNK, W1R), jnp.float32),
        pltpu.VMEM((L,), jnp.float32),
        pltpu.VMEM_SHARED((NP, W1R), jnp.float32),
        pltpu.SemaphoreType.DMA,
        pltpu.SemaphoreType.DMA,
        pltpu.SemaphoreType.DMA,
        pltpu.SemaphoreType.DMA,
        pltpu.SemaphoreType.DMA,
        pltpu.SemaphoreType.DMA,
    ],
  )


def _dense2_body(acc_ref, b1_ref, et_ref, m2_ref, src2_ref, dst2_ref,
                 bs_ref, bd_ref):
    s = acc_ref[0] + acc_ref[1]
    den = jnp.dot(s[:, F1:W1R], et_ref[...],
                  preferred_element_type=jnp.float32)
    o = s[:, 0:F1] / (den + jnp.float32(1e-16)) + b1_ref[...]
    g = jnp.where(o > 0, o, jnp.exp(o) - jnp.float32(1.0))
    p = jnp.dot(g, m2_ref[...], preferred_element_type=jnp.float32)
    lane = lax.broadcasted_iota(jnp.int32, (RPT, L), 1)
    src2_ref[...] = p[:, 0:L] + jnp.where(lane == OUT, 1.0, 0.0)
    dst2_ref[...] = p[:, L:2 * L]
    bs_ref[0, :, :] = jnp.max(p[:, HEADS:L], axis=0, keepdims=True)
    bd_ref[0, :, :] = jnp.max(p[:, L + HEADS:2 * L], axis=0, keepdims=True)


def _dense2(acc1, b1t, Et, M2t):
    grid = NP // RPT
    return pl.pallas_call(
        _dense2_body,
        grid=(grid,),
        in_specs=[
            pl.BlockSpec((NC, RPT, W1R), lambda i: (0, i, 0)),
            pl.BlockSpec((F1,), lambda i: (0,)),
            pl.BlockSpec((L, F1), lambda i: (0, 0)),
            pl.BlockSpec((F1, 2 * L), lambda i: (0, 0)),
        ],
        out_specs=[
            pl.BlockSpec((RPT, L), lambda i: (i, 0)),
            pl.BlockSpec((RPT, L), lambda i: (i, 0)),
            pl.BlockSpec((1, 1, HEADS), lambda i: (i, 0, 0)),
            pl.BlockSpec((1, 1, HEADS), lambda i: (i, 0, 0)),
        ],
        out_shape=[
            jax.ShapeDtypeStruct((NP, L), jnp.float32),
            jax.ShapeDtypeStruct((NP, L), jnp.float32),
            jax.ShapeDtypeStruct((grid, 1, HEADS), jnp.float32),
            jax.ShapeDtypeStruct((grid, 1, HEADS), jnp.float32),
        ],
    )(acc1, b1t, Et, M2t)


def _edge2_body(si3, di3, srct_hbm, dstt_hbm, cv_hbm, z_hbm, out_hbm,
                sidx_all, didx_all, srows0, srows1, drows0, drows1,
                orows0, orows1, cv, acc, gs0, gs1, gd0, gd1, sc0, sc1):
    ci = lax.axis_index("c")
    sid = lax.axis_index("s")
    wid = sid * NC + ci

    pltpu.sync_copy(z_hbm, acc.at[pl.ds(sid * RPT, RPT)])
    pltpu.sync_copy(cv_hbm, cv)
    pltpu.sync_copy(si3.at[wid], sidx_all)
    pltpu.sync_copy(di3.at[wid], didx_all)
    plsc.subcore_barrier()

    srows = (srows0, srows1)
    drows = (drows0, drows1)
    orows = (orows0, orows1)
    gs = (gs0, gs1)
    gd = (gd0, gd1)
    sc = (sc0, sc1)
    cvr = cv[pl.ds(0, L)]

    def start_g(c, b):
        for j in range(S2):
            pltpu.async_copy(srct_hbm.at[sidx_all.at[S2 * c + j]],
                             srows[b].at[pl.ds(j * CHUNK, CHUNK)], gs[b])
            pltpu.async_copy(dstt_hbm.at[didx_all.at[S2 * c + j]],
                             drows[b].at[pl.ds(j * CHUNK, CHUNK)], gd[b])

    def wait_g(c, b):
        for j in range(S2):
            pltpu.make_async_copy(srct_hbm.at[sidx_all.at[S2 * c + j]],
                                  srows[b].at[pl.ds(j * CHUNK, CHUNK)],
                                  gs[b]).wait()
            pltpu.make_async_copy(dstt_hbm.at[didx_all.at[S2 * c + j]],
                                  drows[b].at[pl.ds(j * CHUNK, CHUNK)],
                                  gd[b]).wait()

    def start_s(c, b):
        for j in range(S2):
            pltpu.async_copy(orows[b].at[pl.ds(j * CHUNK, CHUNK)],
                             acc.at[didx_all.at[S2 * c + j]], sc[b], add=True)

    def wait_s(c, b):
        for j in range(S2):
            pltpu.make_async_copy(orows[b].at[pl.ds(j * CHUNK, CHUNK)],
                                  acc.at[didx_all.at[S2 * c + j]],
                                  sc[b]).wait()

    def compute(b):
        sr, dr, orr = srows[b], drows[b], orows[b]

        def edge_body(i, c2):
            s0 = sr[i, pl.ds(0, L)]
            t = s0 + dr[i, pl.ds(0, L)]
            t = jnp.where(t > 0, t, t * jnp.float32(0.2))
            ex = jnp.exp(t - cvr)
            orr[i, pl.ds(0, L)] = s0 * jnp.flip(ex, 0)
            return c2

        lax.fori_loop(0, C2, edge_body, 0, unroll=8)

    start_g(0, 0)
    start_g(1, 1)
    wait_g(0, 0)
    compute(0)
    start_s(0, 0)
    start_g(2, 0)
    wait_g(1, 1)
    compute(1)
    start_s(1, 1)
    start_g(3, 1)

    def pair(k, carry):
        ca = 2 * k
        cb = 2 * k + 1
        wait_g(ca, 0)
        wait_s(ca - 2, 0)
        compute(0)
        start_s(ca, 0)

        @pl.when(ca + 2 < CPT2)
        def _():
            start_g(ca + 2, 0)

        wait_g(cb, 1)
        wait_s(cb - 2, 1)
        compute(1)
        start_s(cb, 1)

        @pl.when(cb + 2 < CPT2)
        def _():
            start_g(cb + 2, 1)

        return carry

    lax.fori_loop(1, (CPT2 - 1) // 2, pair, 0)
    wait_g(CPT2 - 1, 0)
    wait_s(CPT2 - 3, 0)
    compute(0)
    start_s(CPT2 - 1, 0)
    wait_s(CPT2 - 2, 1)
    wait_s(CPT2 - 1, 0)
    plsc.subcore_barrier()
    pltpu.sync_copy(acc.at[pl.ds(sid * RPT, RPT)],
                    out_hbm.at[ci, pl.ds(sid * RPT, RPT)])


@functools.cache
def _edge2():
  return pl.kernel(
    _edge2_body,
    out_type=jax.ShapeDtypeStruct((NC, NP, L), jnp.float32),
    compiler_params=pltpu.CompilerParams(use_tc_tiling_on_sc=False),
    mesh=plsc.VectorSubcoreMesh(core_axis_name="c", subcore_axis_name="s",
                                num_cores=NC, num_subcores=NS),
    scratch_types=[
        pltpu.VMEM((CPT, CHUNK), jnp.int32),
        pltpu.VMEM((CPT, CHUNK), jnp.int32),
        pltpu.VMEM((C2, L), jnp.float32),
        pltpu.VMEM((C2, L), jnp.float32),
        pltpu.VMEM((C2, L), jnp.float32),
        pltpu.VMEM((C2, L), jnp.float32),
        pltpu.VMEM((C2, L), jnp.float32),
        pltpu.VMEM((C2, L), jnp.float32),
        pltpu.VMEM((L,), jnp.float32),
        pltpu.VMEM_SHARED((NP, L), jnp.float32),
        pltpu.SemaphoreType.DMA,
        pltpu.SemaphoreType.DMA,
        pltpu.SemaphoreType.DMA,
        pltpu.SemaphoreType.DMA,
        pltpu.SemaphoreType.DMA,
        pltpu.SemaphoreType.DMA,
    ],
  )


def _final_body(acc_ref, b2_ref, out_ref):
    s = acc_ref[0] + acc_ref[1]
    den = s[:, OUT:OUT + 1]
    out_ref[...] = s / (den + jnp.float32(1e-16)) + b2_ref[...]


def _final(acc2, b2p):
    grid = NP // RPT
    return pl.pallas_call(
        _final_body,
        grid=(grid,),
        in_specs=[
            pl.BlockSpec((NC, RPT, L), lambda i: (0, i, 0)),
            pl.BlockSpec((L,), lambda i: (0,)),
        ],
        out_specs=pl.BlockSpec((RPT, L), lambda i: (i, 0)),
        out_shape=jax.ShapeDtypeStruct((NP, L), jnp.float32),
    )(acc2, b2p)


def _leaky(v):
    return jnp.where(v > 0, v, v * jnp.float32(0.2))


def kernel(x, edge_index, W1, att_src1, att_dst1, b1, W2, att_src2, att_dst2,
           b2):
    f32 = jnp.float32
    i32 = jnp.int32

    loops = jnp.arange(N, dtype=i32)
    padv = jnp.full((EP - ETOT,), N, dtype=i32)
    srcp = jnp.concatenate([edge_index[0].astype(i32), loops, padv])
    dstp = jnp.concatenate([edge_index[1].astype(i32), loops, padv])
    si3 = srcp.reshape(NW, CPT, CHUNK)
    di3 = dstp.reshape(NW, CPT, CHUNK)

    k64 = jnp.arange(F1)
    perm_t = (k64 % HEADS) * HID + k64 // HEADS
    jl = jnp.arange(L)
    head_of = k64 // HID
    A_src = (head_of[:, None] == (jl[None, :] % HEADS)).astype(f32) \
        * att_src1.reshape(F1)[:, None]
    A_dst = (head_of[:, None] == (jl[None, :] % HEADS)).astype(f32) \
        * att_dst1.reshape(F1)[:, None]
    W1f = W1.astype(f32)
    kk = k64 % 32
    hcol = 32 * (k64 // 32) + 16 * (kk % 2) + kk // 2
    M1 = jnp.concatenate([W1f[:, perm_t[hcol]], W1f @ A_src], axis=1)
    WAd = W1f @ A_dst

    x_p = jnp.pad(x.astype(f32), ((0, NP - N), (0, 0)))
    hbf, asil, dstt, bs1, bd1 = _dense1(x_p, M1, WAd)
    srct = jnp.concatenate(
        [lax.bitcast_convert_type(asil, jnp.bfloat16).reshape(NP, 2 * L),
         hbf], axis=1)
    cv1 = _leaky(jnp.max(bs1[:, 0], axis=0) + jnp.max(bd1[:, 0], axis=0))

    z1 = jnp.zeros((RPT, W1R), f32)
    acc1 = _edge1()(si3, di3, srct, dstt, cv1, z1)

    Et = (jl[:, None] == (k64[None, :] % HEADS)).astype(f32)
    w_as2 = (W2 @ att_src2[0]).astype(f32)
    w_ad2 = (W2 @ att_dst2[0]).astype(f32)
    M2 = jnp.concatenate([
        W2.astype(f32), jnp.zeros((F1, L - OUT - HEADS), f32),
        jnp.broadcast_to(w_as2[:, None], (F1, HEADS)),
        jnp.zeros((F1, HEADS), f32),
        jnp.broadcast_to(w_ad2[:, None], (F1, HEADS)),
    ], axis=1)
    M2t = M2[perm_t]
    b1t = b1.astype(f32)[perm_t]

    src2, dst2, bs2, bd2 = _dense2(acc1, b1t, Et, M2t)
    cv2h = _leaky(jnp.max(bs2[:, 0], axis=0) + jnp.max(bd2[:, 0], axis=0))
    cv2 = jnp.concatenate([cv2h, cv2h])

    z2 = jnp.zeros((RPT, L), f32)
    acc2 = _edge2()(si3, di3, src2, dst2, cv2, z2)

    b2p = jnp.zeros((L,), f32).at[:OUT].set(b2.astype(f32))
    outp = _final(acc2, b2p)
    return outp[:N, :OUT]


# speedup vs baseline: 1.0198x; 1.0198x over previous
import functools

import jax
import jax.numpy as jnp
from jax import lax
from jax.experimental import pallas as pl
from jax.experimental.pallas import tpu as pltpu
from jax.experimental.pallas import tpu_sc as plsc

N = 10000
D = 128
HID = 8
HEADS = 8
OUT = 7
E = 320000

NC, NS, L = 2, 16, 16
NW = NC * NS
CHUNK = 128
NP = 10112
RPT = NP // NS
ETOT = E + N
CPT = 82
EP = NW * CHUNK * CPT
S2 = 2
C2 = S2 * CHUNK
CPT2 = CPT // S2

F1 = HEADS * HID
W1R = F1 + L


def _dense1_body(x_ref, m1_ref, wad_ref, hbf_ref, asil_ref, dstt_ref,
                 bs_ref, bd_ref):
    p = jnp.dot(x_ref[...], m1_ref[...], preferred_element_type=jnp.float32)
    q = jnp.dot(x_ref[...], wad_ref[...], preferred_element_type=jnp.float32)
    hbf_ref[...] = p[:, 0:F1].astype(jnp.bfloat16)
    asil_ref[...] = p[:, F1:W1R]
    dstt_ref[...] = q
    bs_ref[0, :, :] = jnp.max(p[:, F1:W1R], axis=0, keepdims=True)
    bd_ref[0, :, :] = jnp.max(q, axis=0, keepdims=True)


def _dense1(x_p, M1, WAd):
    rb = 1264
    grid = NP // rb
    return pl.pallas_call(
        _dense1_body,
        grid=(grid,),
        in_specs=[
            pl.BlockSpec((rb, D), lambda i: (i, 0)),
            pl.BlockSpec((D, W1R), lambda i: (0, 0)),
            pl.BlockSpec((D, L), lambda i: (0, 0)),
        ],
        out_specs=[
            pl.BlockSpec((rb, F1), lambda i: (i, 0)),
            pl.BlockSpec((rb, L), lambda i: (i, 0)),
            pl.BlockSpec((rb, L), lambda i: (i, 0)),
            pl.BlockSpec((1, 1, L), lambda i: (i, 0, 0)),
            pl.BlockSpec((1, 1, L), lambda i: (i, 0, 0)),
        ],
        out_shape=[
            jax.ShapeDtypeStruct((NP, F1), jnp.bfloat16),
            jax.ShapeDtypeStruct((NP, L), jnp.float32),
            jax.ShapeDtypeStruct((NP, L), jnp.float32),
            jax.ShapeDtypeStruct((grid, 1, L), jnp.float32),
            jax.ShapeDtypeStruct((grid, 1, L), jnp.float32),
        ],
    )(x_p, M1, WAd)


def _edge1_body(si3, di3, srct_hbm, dstt_hbm, cv_hbm, z_hbm, out_hbm,
                sidx_all, didx_all, srows0, srows1, drows0, drows1,
                orows0, orows1, cv, acc, gs0, gs1, gd0, gd1, sc0, sc1):
    ci = lax.axis_index("c")
    sid = lax.axis_index("s")
    wid = sid * NC + ci

    pltpu.sync_copy(z_hbm, acc.at[pl.ds(sid * RPT, RPT)])
    pltpu.sync_copy(cv_hbm, cv)
    pltpu.sync_copy(si3.at[wid], sidx_all)
    pltpu.sync_copy(di3.at[wid], didx_all)
    plsc.subcore_barrier()

    srows = (srows0, srows1)
    drows = (drows0, drows1)
    orows = (orows0, orows1)
    gs = (gs0, gs1)
    gd = (gd0, gd1)
    sc = (sc0, sc1)
    cvr = cv[pl.ds(0, L)]

    def start_g(c, b):
        pltpu.async_copy(srct_hbm.at[sidx_all.at[c]], srows[b], gs[b])
        pltpu.async_copy(dstt_hbm.at[didx_all.at[c]], drows[b], gd[b])

    def wait_g(c, b):
        pltpu.make_async_copy(srct_hbm.at[sidx_all.at[c]], srows[b],
                              gs[b]).wait()
        pltpu.make_async_copy(dstt_hbm.at[didx_all.at[c]], drows[b],
                              gd[b]).wait()

    def start_s(c, b):
        pltpu.async_copy(orows[b], acc.at[didx_all.at[c]], sc[b], add=True)

    def wait_s(c, b):
        pltpu.make_async_copy(orows[b], acc.at[didx_all.at[c]], sc[b]).wait()

    def compute(b):
        sr, dr, orr = srows[b], drows[b], orows[b]

        def edge_body(i, c2):
            asil = plsc.bitcast(sr[i, pl.ds(0, 2 * L)], jnp.float32)
            t = asil + dr[i, pl.ds(0, L)]
            t = jnp.where(t > 0, t, t * jnp.float32(0.2))
            ex = jnp.exp(t - cvr)
            for v in range(2):
                hv = sr[i, pl.ds(2 * L + 2 * v * L, 2 * L)]
                ha, hb = plsc.unpack(hv, format=plsc.PackFormat.INTERLEAVED)
                orr[i, pl.ds(2 * v * L, L)] = ha * ex
                orr[i, pl.ds(2 * v * L + L, L)] = hb * ex
            orr[i, pl.ds(F1, L)] = ex
            return c2

        lax.fori_loop(0, CHUNK, edge_body, 0, unroll=4)

    start_g(0, 0)
    start_g(1, 1)
    wait_g(0, 0)
    compute(0)
    start_s(0, 0)
    start_g(2, 0)
    wait_g(1, 1)
    compute(1)
    start_s(1, 1)
    start_g(3, 1)

    def pair(k, carry):
        ca = 2 * k
        cb = 2 * k + 1
        wait_g(ca, 0)
        wait_s(ca - 2, 0)
        compute(0)
        start_s(ca, 0)

        @pl.when(ca + 2 < CPT)
        def _():
            start_g(ca + 2, 0)

        wait_g(cb, 1)
        wait_s(cb - 2, 1)
        compute(1)
        start_s(cb, 1)

        @pl.when(cb + 2 < CPT)
        def _():
            start_g(cb + 2, 1)

        return carry

    lax.fori_loop(1, CPT // 2, pair, 0)
    wait_s(CPT - 2, 0)
    wait_s(CPT - 1, 1)
    plsc.subcore_barrier()
    pltpu.sync_copy(acc.at[pl.ds(sid * RPT, RPT)],
                    out_hbm.at[ci, pl.ds(sid * RPT, RPT)])


@functools.cache
def _edge1():
  return pl.kernel(
    _edge1_body,
    out_type=jax.ShapeDtypeStruct((NC, NP, W1R), jnp.float32),
    compiler_params=pltpu.CompilerParams(use_tc_tiling_on_sc=False,
                                         needs_layout_passes=False),
    mesh=plsc.VectorSubcoreMesh(core_axis_name="c", subcore_axis_name="s",
                                num_cores=NC, num_subcores=NS),
    scratch_types=[
        pltpu.VMEM((CPT, CHUNK), jnp.int32),
        pltpu.VMEM((CPT, CHUNK), jnp.int32),
        pltpu.VMEM((CHUNK, 6 * L), jnp.bfloat16),
        pltpu.VMEM((CHUNK, 6 * L), jnp.bfloat16),
        pltpu.VMEM((CHUNK, L), jnp.float32),
        pltpu.VMEM((CHUNK, L), jnp.float32),
        pltpu.VMEM((CHUNK, W1R), jnp.float32),
        pltpu.VMEM((CHUNK, W1R), jnp.float32),
        pltpu.VMEM((L,), jnp.float32),
        pltpu.VMEM_SHARED((NP, W1R), jnp.float32),
        pltpu.SemaphoreType.DMA,
        pltpu.SemaphoreType.DMA,
        pltpu.SemaphoreType.DMA,
        pltpu.SemaphoreType.DMA,
        pltpu.SemaphoreType.DMA,
        pltpu.SemaphoreType.DMA,
    ],
  )


def _dense2_body(acc_ref, b1_ref, et_ref, m2_ref, src2_ref, dst2_ref,
                 bs_ref, bd_ref):
    s = acc_ref[0] + acc_ref[1]
    den = jnp.dot(s[:, F1:W1R], et_ref[...],
                  preferred_element_type=jnp.float32)
    o = s[:, 0:F1] / (den + jnp.float32(1e-16)) + b1_ref[...]
    g = jnp.where(o > 0, o, jnp.exp(o) - jnp.float32(1.0))
    p = jnp.dot(g, m2_ref[...], preferred_element_type=jnp.float32)
    lane = lax.broadcasted_iota(jnp.int32, (RPT, 2 * L), 1)
    src2_ref[...] = p[:, 0:2 * L] + jnp.where(lane == OUT, 1.0, 0.0)
    dst2_ref[...] = p[:, 2 * L:3 * L]
    bs_ref[0, :, :] = jnp.max(p[:, L:2 * L], axis=0, keepdims=True)
    bd_ref[0, :, :] = jnp.max(p[:, 2 * L:3 * L], axis=0, keepdims=True)


def _dense2(acc1, b1t, Et, M2t):
    grid = NP // RPT
    return pl.pallas_call(
        _dense2_body,
        grid=(grid,),
        in_specs=[
            pl.BlockSpec((NC, RPT, W1R), lambda i: (0, i, 0)),
            pl.BlockSpec((F1,), lambda i: (0,)),
            pl.BlockSpec((L, F1), lambda i: (0, 0)),
            pl.BlockSpec((F1, 3 * L), lambda i: (0, 0)),
        ],
        out_specs=[
            pl.BlockSpec((RPT, 2 * L), lambda i: (i, 0)),
            pl.BlockSpec((RPT, L), lambda i: (i, 0)),
            pl.BlockSpec((1, 1, L), lambda i: (i, 0, 0)),
            pl.BlockSpec((1, 1, L), lambda i: (i, 0, 0)),
        ],
        out_shape=[
            jax.ShapeDtypeStruct((NP, 2 * L), jnp.float32),
            jax.ShapeDtypeStruct((NP, L), jnp.float32),
            jax.ShapeDtypeStruct((grid, 1, L), jnp.float32),
            jax.ShapeDtypeStruct((grid, 1, L), jnp.float32),
        ],
    )(acc1, b1t, Et, M2t)


def _edge2_body(si3, di3, srct_hbm, dstt_hbm, cv_hbm, z_hbm, out_hbm,
                sidx_all, didx_all, srows0, srows1, drows0, drows1,
                orows0, orows1, cv, acc, gs0, gs1, gd0, gd1, sc0, sc1):
    ci = lax.axis_index("c")
    sid = lax.axis_index("s")
    wid = sid * NC + ci

    pltpu.sync_copy(z_hbm, acc.at[pl.ds(sid * RPT, RPT)])
    pltpu.sync_copy(cv_hbm, cv)
    pltpu.sync_copy(si3.at[wid], sidx_all)
    pltpu.sync_copy(di3.at[wid], didx_all)
    plsc.subcore_barrier()

    srows = (srows0, srows1)
    drows = (drows0, drows1)
    orows = (orows0, orows1)
    gs = (gs0, gs1)
    gd = (gd0, gd1)
    sc = (sc0, sc1)
    cvr = cv[pl.ds(0, L)]

    def start_g(c, b):
        for j in range(S2):
            pltpu.async_copy(srct_hbm.at[sidx_all.at[S2 * c + j]],
                             srows[b].at[pl.ds(j * CHUNK, CHUNK)], gs[b])
            pltpu.async_copy(dstt_hbm.at[didx_all.at[S2 * c + j]],
                             drows[b].at[pl.ds(j * CHUNK, CHUNK)], gd[b])

    def wait_g(c, b):
        for j in range(S2):
            pltpu.make_async_copy(srct_hbm.at[sidx_all.at[S2 * c + j]],
                                  srows[b].at[pl.ds(j * CHUNK, CHUNK)],
                                  gs[b]).wait()
            pltpu.make_async_copy(dstt_hbm.at[didx_all.at[S2 * c + j]],
                                  drows[b].at[pl.ds(j * CHUNK, CHUNK)],
                                  gd[b]).wait()

    def start_s(c, b):
        for j in range(S2):
            pltpu.async_copy(orows[b].at[pl.ds(j * CHUNK, CHUNK)],
                             acc.at[didx_all.at[S2 * c + j]], sc[b], add=True)

    def wait_s(c, b):
        for j in range(S2):
            pltpu.make_async_copy(orows[b].at[pl.ds(j * CHUNK, CHUNK)],
                                  acc.at[didx_all.at[S2 * c + j]],
                                  sc[b]).wait()

    def compute(b):
        sr, dr, orr = srows[b], drows[b], orows[b]

        def edge_body(i, c2):
            s0 = sr[i, pl.ds(0, L)]
            s1 = sr[i, pl.ds(L, L)]
            dd = dr[i, pl.ds(0, L)]
            t = s1 + dd
            t = jnp.where(t > 0, t, t * jnp.float32(0.2))
            ex = jnp.exp(t - cvr)
            orr[i, pl.ds(0, L)] = s0 * ex
            return c2

        lax.fori_loop(0, C2, edge_body, 0, unroll=8)

    start_g(0, 0)
    start_g(1, 1)
    wait_g(0, 0)
    compute(0)
    start_s(0, 0)
    start_g(2, 0)
    wait_g(1, 1)
    compute(1)
    start_s(1, 1)
    start_g(3, 1)

    def pair(k, carry):
        ca = 2 * k
        cb = 2 * k + 1
        wait_g(ca, 0)
        wait_s(ca - 2, 0)
        compute(0)
        start_s(ca, 0)

        @pl.when(ca + 2 < CPT2)
        def _():
            start_g(ca + 2, 0)

        wait_g(cb, 1)
        wait_s(cb - 2, 1)
        compute(1)
        start_s(cb, 1)

        @pl.when(cb + 2 < CPT2)
        def _():
            start_g(cb + 2, 1)

        return carry

    lax.fori_loop(1, (CPT2 - 1) // 2, pair, 0)
    wait_g(CPT2 - 1, 0)
    wait_s(CPT2 - 3, 0)
    compute(0)
    start_s(CPT2 - 1, 0)
    wait_s(CPT2 - 2, 1)
    wait_s(CPT2 - 1, 0)
    plsc.subcore_barrier()
    pltpu.sync_copy(acc.at[pl.ds(sid * RPT, RPT)],
                    out_hbm.at[ci, pl.ds(sid * RPT, RPT)])


@functools.cache
def _edge2():
  return pl.kernel(
    _edge2_body,
    out_type=jax.ShapeDtypeStruct((NC, NP, L), jnp.float32),
    compiler_params=pltpu.CompilerParams(use_tc_tiling_on_sc=False),
    mesh=plsc.VectorSubcoreMesh(core_axis_name="c", subcore_axis_name="s",
                                num_cores=NC, num_subcores=NS),
    scratch_types=[
        pltpu.VMEM((CPT, CHUNK), jnp.int32),
        pltpu.VMEM((CPT, CHUNK), jnp.int32),
        pltpu.VMEM((C2, 2 * L), jnp.float32),
        pltpu.VMEM((C2, 2 * L), jnp.float32),
        pltpu.VMEM((C2, L), jnp.float32),
        pltpu.VMEM((C2, L), jnp.float32),
        pltpu.VMEM((C2, L), jnp.float32),
        pltpu.VMEM((C2, L), jnp.float32),
        pltpu.VMEM((L,), jnp.float32),
        pltpu.VMEM_SHARED((NP, L), jnp.float32),
        pltpu.SemaphoreType.DMA,
        pltpu.SemaphoreType.DMA,
        pltpu.SemaphoreType.DMA,
        pltpu.SemaphoreType.DMA,
        pltpu.SemaphoreType.DMA,
        pltpu.SemaphoreType.DMA,
    ],
  )


def _final_body(acc_ref, b2_ref, out_ref):
    s = acc_ref[0] + acc_ref[1]
    den = s[:, OUT:OUT + 1]
    out_ref[...] = s / (den + jnp.float32(1e-16)) + b2_ref[...]


def _final(acc2, b2p):
    grid = NP // RPT
    return pl.pallas_call(
        _final_body,
        grid=(grid,),
        in_specs=[
            pl.BlockSpec((NC, RPT, L), lambda i: (0, i, 0)),
            pl.BlockSpec((L,), lambda i: (0,)),
        ],
        out_specs=pl.BlockSpec((RPT, L), lambda i: (i, 0)),
        out_shape=jax.ShapeDtypeStruct((NP, L), jnp.float32),
    )(acc2, b2p)


def _leaky(v):
    return jnp.where(v > 0, v, v * jnp.float32(0.2))


def kernel(x, edge_index, W1, att_src1, att_dst1, b1, W2, att_src2, att_dst2,
           b2):
    f32 = jnp.float32
    i32 = jnp.int32

    loops = jnp.arange(N, dtype=i32)
    padv = jnp.full((EP - ETOT,), N, dtype=i32)
    srcp = jnp.concatenate([edge_index[0].astype(i32), loops, padv])
    dstp = jnp.concatenate([edge_index[1].astype(i32), loops, padv])
    si3 = srcp.reshape(NW, CPT, CHUNK)
    di3 = dstp.reshape(NW, CPT, CHUNK)

    k64 = jnp.arange(F1)
    perm_t = (k64 % HEADS) * HID + k64 // HEADS
    jl = jnp.arange(L)
    head_of = k64 // HID
    A_src = (head_of[:, None] == (jl[None, :] % HEADS)).astype(f32) \
        * att_src1.reshape(F1)[:, None]
    A_dst = (head_of[:, None] == (jl[None, :] % HEADS)).astype(f32) \
        * att_dst1.reshape(F1)[:, None]
    W1f = W1.astype(f32)
    kk = k64 % 32
    hcol = 32 * (k64 // 32) + 16 * (kk % 2) + kk // 2
    M1 = jnp.concatenate([W1f[:, perm_t[hcol]], W1f @ A_src], axis=1)
    WAd = W1f @ A_dst

    x_p = jnp.pad(x.astype(f32), ((0, NP - N), (0, 0)))
    hbf, asil, dstt, bs1, bd1 = _dense1(x_p, M1, WAd)
    srct = jnp.concatenate(
        [lax.bitcast_convert_type(asil, jnp.bfloat16).reshape(NP, 2 * L),
         hbf], axis=1)
    cv1 = _leaky(jnp.max(bs1[:, 0], axis=0) + jnp.max(bd1[:, 0], axis=0))

    z1 = jnp.zeros((RPT, W1R), f32)
    acc1 = _edge1()(si3, di3, srct, dstt, cv1, z1)

    Et = (jl[:, None] == (k64[None, :] % HEADS)).astype(f32)
    w_as2 = (W2 @ att_src2[0]).astype(f32)
    w_ad2 = (W2 @ att_dst2[0]).astype(f32)
    M2 = jnp.concatenate([
        W2.astype(f32), jnp.zeros((F1, L - OUT), f32),
        jnp.broadcast_to(w_as2[:, None], (F1, L)),
        jnp.broadcast_to(w_ad2[:, None], (F1, L)),
    ], axis=1)
    M2t = M2[perm_t]
    b1t = b1.astype(f32)[perm_t]

    src2, dst2, bs2, bd2 = _dense2(acc1, b1t, Et, M2t)
    cv2 = _leaky(jnp.max(bs2[:, 0], axis=0) + jnp.max(bd2[:, 0], axis=0))

    z2 = jnp.zeros((RPT, L), f32)
    acc2 = _edge2()(si3, di3, src2, dst2, cv2, z2)

    b2p = jnp.zeros((L,), f32).at[:OUT].set(b2.astype(f32))
    outp = _final(acc2, b2p)
    return outp[:N, :OUT]


# speedup vs baseline: 1.4573x; 1.4290x over previous
import functools

import jax
import jax.numpy as jnp
from jax import lax
from jax.experimental import pallas as pl
from jax.experimental.pallas import tpu as pltpu
from jax.experimental.pallas import tpu_sc as plsc

N = 10000
D = 128
HID = 8
HEADS = 8
OUT = 7
E = 320000

NC, NS, L = 2, 16, 16
NW = NC * NS
CHUNK = 128
NP = 10112
RPT = NP // NS
ETOT = E + N
CPT = 82
EP = NW * CHUNK * CPT
S2 = 2
C2 = S2 * CHUNK
CPT2 = CPT // S2

F1 = HEADS * HID
W1R = F1 + L


def _dense1_body(x_ref, m1_ref, wad_ref, hbf_ref, asil_ref, dstt_ref,
                 bs_ref, bd_ref):
    p = jnp.dot(x_ref[...], m1_ref[...], preferred_element_type=jnp.float32)
    q = jnp.dot(x_ref[...], wad_ref[...], preferred_element_type=jnp.float32)
    hbf_ref[...] = p[:, 0:F1].astype(jnp.bfloat16)
    asil_ref[...] = p[:, F1:W1R]
    dstt_ref[...] = q
    bs_ref[0, :, :] = jnp.max(p[:, F1:W1R], axis=0, keepdims=True)
    bd_ref[0, :, :] = jnp.max(q, axis=0, keepdims=True)


def _dense1(x_p, M1, WAd):
    rb = 1264
    grid = NP // rb
    return pl.pallas_call(
        _dense1_body,
        grid=(grid,),
        in_specs=[
            pl.BlockSpec((rb, D), lambda i: (i, 0)),
            pl.BlockSpec((D, W1R), lambda i: (0, 0)),
            pl.BlockSpec((D, L), lambda i: (0, 0)),
        ],
        out_specs=[
            pl.BlockSpec((rb, F1), lambda i: (i, 0)),
            pl.BlockSpec((rb, L), lambda i: (i, 0)),
            pl.BlockSpec((rb, L), lambda i: (i, 0)),
            pl.BlockSpec((1, 1, L), lambda i: (i, 0, 0)),
            pl.BlockSpec((1, 1, L), lambda i: (i, 0, 0)),
        ],
        out_shape=[
            jax.ShapeDtypeStruct((NP, F1), jnp.bfloat16),
            jax.ShapeDtypeStruct((NP, L), jnp.float32),
            jax.ShapeDtypeStruct((NP, L), jnp.float32),
            jax.ShapeDtypeStruct((grid, 1, L), jnp.float32),
            jax.ShapeDtypeStruct((grid, 1, L), jnp.float32),
        ],
    )(x_p, M1, WAd)


def _edge1_body(si3, di3, srct_hbm, dstt_hbm, cv_hbm, z_hbm, out_hbm,
                sidx_all, didx_all, srows0, srows1, drows0, drows1,
                orows0, orows1, cv, acc, gs0, gs1, gd0, gd1, sc0, sc1):
    ci = lax.axis_index("c")
    sid = lax.axis_index("s")
    wid = sid * NC + ci

    pltpu.sync_copy(z_hbm, acc.at[pl.ds(sid * RPT, RPT)])
    pltpu.sync_copy(cv_hbm, cv)
    pltpu.sync_copy(si3.at[wid], sidx_all)
    pltpu.sync_copy(di3.at[wid], didx_all)
    plsc.subcore_barrier()

    srows = (srows0, srows1)
    drows = (drows0, drows1)
    orows = (orows0, orows1)
    gs = (gs0, gs1)
    gd = (gd0, gd1)
    sc = (sc0, sc1)
    cvr = cv[pl.ds(0, L)]

    def start_g(c, b):
        pltpu.async_copy(srct_hbm.at[sidx_all.at[c]], srows[b], gs[b])
        pltpu.async_copy(dstt_hbm.at[didx_all.at[c]], drows[b], gd[b])

    def wait_g(c, b):
        pltpu.make_async_copy(srct_hbm.at[sidx_all.at[c]], srows[b],
                              gs[b]).wait()
        pltpu.make_async_copy(dstt_hbm.at[didx_all.at[c]], drows[b],
                              gd[b]).wait()

    def start_s(c, b):
        pltpu.async_copy(orows[b], acc.at[didx_all.at[c]], sc[b], add=True)

    def wait_s(c, b):
        pltpu.make_async_copy(orows[b], acc.at[didx_all.at[c]], sc[b]).wait()

    def compute(b):
        sr, dr, orr = srows[b], drows[b], orows[b]

        @plsc.parallel_loop(0, CHUNK, step=1, unroll=4)
        def edge_body(i):
            asil = plsc.bitcast(sr[i, pl.ds(0, 2 * L)], jnp.float32)
            t = asil + dr[i, pl.ds(0, L)]
            t = jnp.where(t > 0, t, t * jnp.float32(0.2))
            ex = jnp.exp(t - cvr)
            for v in range(2):
                hv = sr[i, pl.ds(2 * L + 2 * v * L, 2 * L)]
                ha, hb = plsc.unpack(hv, format=plsc.PackFormat.INTERLEAVED)
                orr[i, pl.ds(2 * v * L, L)] = ha * ex
                orr[i, pl.ds(2 * v * L + L, L)] = hb * ex
            orr[i, pl.ds(F1, L)] = ex

    start_g(0, 0)
    start_g(1, 1)
    wait_g(0, 0)
    compute(0)
    start_s(0, 0)
    start_g(2, 0)
    wait_g(1, 1)
    compute(1)
    start_s(1, 1)
    start_g(3, 1)

    def pair(k, carry):
        ca = 2 * k
        cb = 2 * k + 1
        wait_g(ca, 0)
        wait_s(ca - 2, 0)
        compute(0)
        start_s(ca, 0)

        @pl.when(ca + 2 < CPT)
        def _():
            start_g(ca + 2, 0)

        wait_g(cb, 1)
        wait_s(cb - 2, 1)
        compute(1)
        start_s(cb, 1)

        @pl.when(cb + 2 < CPT)
        def _():
            start_g(cb + 2, 1)

        return carry

    lax.fori_loop(1, CPT // 2, pair, 0)
    wait_s(CPT - 2, 0)
    wait_s(CPT - 1, 1)
    plsc.subcore_barrier()
    pltpu.sync_copy(acc.at[pl.ds(sid * RPT, RPT)],
                    out_hbm.at[ci, pl.ds(sid * RPT, RPT)])


@functools.cache
def _edge1():
  return pl.kernel(
    _edge1_body,
    out_type=jax.ShapeDtypeStruct((NC, NP, W1R), jnp.float32),
    compiler_params=pltpu.CompilerParams(use_tc_tiling_on_sc=False,
                                         needs_layout_passes=False),
    mesh=plsc.VectorSubcoreMesh(core_axis_name="c", subcore_axis_name="s",
                                num_cores=NC, num_subcores=NS),
    scratch_types=[
        pltpu.VMEM((CPT, CHUNK), jnp.int32),
        pltpu.VMEM((CPT, CHUNK), jnp.int32),
        pltpu.VMEM((CHUNK, 6 * L), jnp.bfloat16),
        pltpu.VMEM((CHUNK, 6 * L), jnp.bfloat16),
        pltpu.VMEM((CHUNK, L), jnp.float32),
        pltpu.VMEM((CHUNK, L), jnp.float32),
        pltpu.VMEM((CHUNK, W1R), jnp.float32),
        pltpu.VMEM((CHUNK, W1R), jnp.float32),
        pltpu.VMEM((L,), jnp.float32),
        pltpu.VMEM_SHARED((NP, W1R), jnp.float32),
        pltpu.SemaphoreType.DMA,
        pltpu.SemaphoreType.DMA,
        pltpu.SemaphoreType.DMA,
        pltpu.SemaphoreType.DMA,
        pltpu.SemaphoreType.DMA,
        pltpu.SemaphoreType.DMA,
    ],
  )


def _dense2_body(acc_ref, b1_ref, et_ref, m2_ref, src2_ref, dst2_ref,
                 bs_ref, bd_ref):
    s = acc_ref[0] + acc_ref[1]
    den = jnp.dot(s[:, F1:W1R], et_ref[...],
                  preferred_element_type=jnp.float32)
    o = s[:, 0:F1] / (den + jnp.float32(1e-16)) + b1_ref[...]
    g = jnp.where(o > 0, o, jnp.exp(o) - jnp.float32(1.0))
    p = jnp.dot(g, m2_ref[...], preferred_element_type=jnp.float32)
    lane = lax.broadcasted_iota(jnp.int32, (RPT, 2 * L), 1)
    src2_ref[...] = p[:, 0:2 * L] + jnp.where(lane == OUT, 1.0, 0.0)
    dst2_ref[...] = p[:, 2 * L:3 * L]
    bs_ref[0, :, :] = jnp.max(p[:, L:2 * L], axis=0, keepdims=True)
    bd_ref[0, :, :] = jnp.max(p[:, 2 * L:3 * L], axis=0, keepdims=True)


def _dense2(acc1, b1t, Et, M2t):
    grid = NP // RPT
    return pl.pallas_call(
        _dense2_body,
        grid=(grid,),
        in_specs=[
            pl.BlockSpec((NC, RPT, W1R), lambda i: (0, i, 0)),
            pl.BlockSpec((F1,), lambda i: (0,)),
            pl.BlockSpec((L, F1), lambda i: (0, 0)),
            pl.BlockSpec((F1, 3 * L), lambda i: (0, 0)),
        ],
        out_specs=[
            pl.BlockSpec((RPT, 2 * L), lambda i: (i, 0)),
            pl.BlockSpec((RPT, L), lambda i: (i, 0)),
            pl.BlockSpec((1, 1, L), lambda i: (i, 0, 0)),
            pl.BlockSpec((1, 1, L), lambda i: (i, 0, 0)),
        ],
        out_shape=[
            jax.ShapeDtypeStruct((NP, 2 * L), jnp.float32),
            jax.ShapeDtypeStruct((NP, L), jnp.float32),
            jax.ShapeDtypeStruct((grid, 1, L), jnp.float32),
            jax.ShapeDtypeStruct((grid, 1, L), jnp.float32),
        ],
    )(acc1, b1t, Et, M2t)


def _edge2_body(si3, di3, srct_hbm, dstt_hbm, cv_hbm, z_hbm, out_hbm,
                sidx_all, didx_all, srows0, srows1, drows0, drows1,
                orows0, orows1, cv, acc, gs0, gs1, gd0, gd1, sc0, sc1):
    ci = lax.axis_index("c")
    sid = lax.axis_index("s")
    wid = sid * NC + ci

    pltpu.sync_copy(z_hbm, acc.at[pl.ds(sid * RPT, RPT)])
    pltpu.sync_copy(cv_hbm, cv)
    pltpu.sync_copy(si3.at[wid], sidx_all)
    pltpu.sync_copy(di3.at[wid], didx_all)
    plsc.subcore_barrier()

    srows = (srows0, srows1)
    drows = (drows0, drows1)
    orows = (orows0, orows1)
    gs = (gs0, gs1)
    gd = (gd0, gd1)
    sc = (sc0, sc1)
    cvr = cv[pl.ds(0, L)]

    def start_g(c, b):
        for j in range(S2):
            pltpu.async_copy(srct_hbm.at[sidx_all.at[S2 * c + j]],
                             srows[b].at[pl.ds(j * CHUNK, CHUNK)], gs[b])
            pltpu.async_copy(dstt_hbm.at[didx_all.at[S2 * c + j]],
                             drows[b].at[pl.ds(j * CHUNK, CHUNK)], gd[b])

    def wait_g(c, b):
        for j in range(S2):
            pltpu.make_async_copy(srct_hbm.at[sidx_all.at[S2 * c + j]],
                                  srows[b].at[pl.ds(j * CHUNK, CHUNK)],
                                  gs[b]).wait()
            pltpu.make_async_copy(dstt_hbm.at[didx_all.at[S2 * c + j]],
                                  drows[b].at[pl.ds(j * CHUNK, CHUNK)],
                                  gd[b]).wait()

    def start_s(c, b):
        for j in range(S2):
            pltpu.async_copy(orows[b].at[pl.ds(j * CHUNK, CHUNK)],
                             acc.at[didx_all.at[S2 * c + j]], sc[b], add=True)

    def wait_s(c, b):
        for j in range(S2):
            pltpu.make_async_copy(orows[b].at[pl.ds(j * CHUNK, CHUNK)],
                                  acc.at[didx_all.at[S2 * c + j]],
                                  sc[b]).wait()

    def compute(b):
        sr, dr, orr = srows[b], drows[b], orows[b]

        @plsc.parallel_loop(0, C2, step=1, unroll=8)
        def edge_body(i):
            s0 = sr[i, pl.ds(0, L)]
            s1 = sr[i, pl.ds(L, L)]
            dd = dr[i, pl.ds(0, L)]
            t = s1 + dd
            t = jnp.where(t > 0, t, t * jnp.float32(0.2))
            ex = jnp.exp(t - cvr)
            orr[i, pl.ds(0, L)] = s0 * ex

    start_g(0, 0)
    start_g(1, 1)
    wait_g(0, 0)
    compute(0)
    start_s(0, 0)
    start_g(2, 0)
    wait_g(1, 1)
    compute(1)
    start_s(1, 1)
    start_g(3, 1)

    def pair(k, carry):
        ca = 2 * k
        cb = 2 * k + 1
        wait_g(ca, 0)
        wait_s(ca - 2, 0)
        compute(0)
        start_s(ca, 0)

        @pl.when(ca + 2 < CPT2)
        def _():
            start_g(ca + 2, 0)

        wait_g(cb, 1)
        wait_s(cb - 2, 1)
        compute(1)
        start_s(cb, 1)

        @pl.when(cb + 2 < CPT2)
        def _():
            start_g(cb + 2, 1)

        return carry

    lax.fori_loop(1, (CPT2 - 1) // 2, pair, 0)
    wait_g(CPT2 - 1, 0)
    wait_s(CPT2 - 3, 0)
    compute(0)
    start_s(CPT2 - 1, 0)
    wait_s(CPT2 - 2, 1)
    wait_s(CPT2 - 1, 0)
    plsc.subcore_barrier()
    pltpu.sync_copy(acc.at[pl.ds(sid * RPT, RPT)],
                    out_hbm.at[ci, pl.ds(sid * RPT, RPT)])


@functools.cache
def _edge2():
  return pl.kernel(
    _edge2_body,
    out_type=jax.ShapeDtypeStruct((NC, NP, L), jnp.float32),
    compiler_params=pltpu.CompilerParams(use_tc_tiling_on_sc=False),
    mesh=plsc.VectorSubcoreMesh(core_axis_name="c", subcore_axis_name="s",
                                num_cores=NC, num_subcores=NS),
    scratch_types=[
        pltpu.VMEM((CPT, CHUNK), jnp.int32),
        pltpu.VMEM((CPT, CHUNK), jnp.int32),
        pltpu.VMEM((C2, 2 * L), jnp.float32),
        pltpu.VMEM((C2, 2 * L), jnp.float32),
        pltpu.VMEM((C2, L), jnp.float32),
        pltpu.VMEM((C2, L), jnp.float32),
        pltpu.VMEM((C2, L), jnp.float32),
        pltpu.VMEM((C2, L), jnp.float32),
        pltpu.VMEM((L,), jnp.float32),
        pltpu.VMEM_SHARED((NP, L), jnp.float32),
        pltpu.SemaphoreType.DMA,
        pltpu.SemaphoreType.DMA,
        pltpu.SemaphoreType.DMA,
        pltpu.SemaphoreType.DMA,
        pltpu.SemaphoreType.DMA,
        pltpu.SemaphoreType.DMA,
    ],
  )


def _final_body(acc_ref, b2_ref, out_ref):
    s = acc_ref[0] + acc_ref[1]
    den = s[:, OUT:OUT + 1]
    out_ref[...] = s / (den + jnp.float32(1e-16)) + b2_ref[...]


def _final(acc2, b2p):
    grid = NP // RPT
    return pl.pallas_call(
        _final_body,
        grid=(grid,),
        in_specs=[
            pl.BlockSpec((NC, RPT, L), lambda i: (0, i, 0)),
            pl.BlockSpec((L,), lambda i: (0,)),
        ],
        out_specs=pl.BlockSpec((RPT, L), lambda i: (i, 0)),
        out_shape=jax.ShapeDtypeStruct((NP, L), jnp.float32),
    )(acc2, b2p)


def _leaky(v):
    return jnp.where(v > 0, v, v * jnp.float32(0.2))


def kernel(x, edge_index, W1, att_src1, att_dst1, b1, W2, att_src2, att_dst2,
           b2):
    f32 = jnp.float32
    i32 = jnp.int32

    loops = jnp.arange(N, dtype=i32)
    padv = jnp.full((EP - ETOT,), N, dtype=i32)
    srcp = jnp.concatenate([edge_index[0].astype(i32), loops, padv])
    dstp = jnp.concatenate([edge_index[1].astype(i32), loops, padv])
    si3 = srcp.reshape(NW, CPT, CHUNK)
    di3 = dstp.reshape(NW, CPT, CHUNK)

    k64 = jnp.arange(F1)
    perm_t = (k64 % HEADS) * HID + k64 // HEADS
    jl = jnp.arange(L)
    head_of = k64 // HID
    A_src = (head_of[:, None] == (jl[None, :] % HEADS)).astype(f32) \
        * att_src1.reshape(F1)[:, None]
    A_dst = (head_of[:, None] == (jl[None, :] % HEADS)).astype(f32) \
        * att_dst1.reshape(F1)[:, None]
    W1f = W1.astype(f32)
    kk = k64 % 32
    hcol = 32 * (k64 // 32) + 16 * (kk % 2) + kk // 2
    M1 = jnp.concatenate([W1f[:, perm_t[hcol]], W1f @ A_src], axis=1)
    WAd = W1f @ A_dst

    x_p = jnp.pad(x.astype(f32), ((0, NP - N), (0, 0)))
    hbf, asil, dstt, bs1, bd1 = _dense1(x_p, M1, WAd)
    srct = jnp.concatenate(
        [lax.bitcast_convert_type(asil, jnp.bfloat16).reshape(NP, 2 * L),
         hbf], axis=1)
    cv1 = _leaky(jnp.max(bs1[:, 0], axis=0) + jnp.max(bd1[:, 0], axis=0))

    z1 = jnp.zeros((RPT, W1R), f32)
    acc1 = _edge1()(si3, di3, srct, dstt, cv1, z1)

    Et = (jl[:, None] == (k64[None, :] % HEADS)).astype(f32)
    w_as2 = (W2 @ att_src2[0]).astype(f32)
    w_ad2 = (W2 @ att_dst2[0]).astype(f32)
    M2 = jnp.concatenate([
        W2.astype(f32), jnp.zeros((F1, L - OUT), f32),
        jnp.broadcast_to(w_as2[:, None], (F1, L)),
        jnp.broadcast_to(w_ad2[:, None], (F1, L)),
    ], axis=1)
    M2t = M2[perm_t]
    b1t = b1.astype(f32)[perm_t]

    src2, dst2, bs2, bd2 = _dense2(acc1, b1t, Et, M2t)
    cv2 = _leaky(jnp.max(bs2[:, 0], axis=0) + jnp.max(bd2[:, 0], axis=0))

    z2 = jnp.zeros((RPT, L), f32)
    acc2 = _edge2()(si3, di3, src2, dst2, cv2, z2)

    b2p = jnp.zeros((L,), f32).at[:OUT].set(b2.astype(f32))
    outp = _final(acc2, b2p)
    return outp[:N, :OUT]
